# baseline (speedup 1.0000x reference)
import functools

import jax
import jax.numpy as jnp
from jax import lax
from jax.experimental import pallas as pl
from jax.experimental.pallas import tpu as pltpu
from jax.experimental.pallas import tpu_sc as plsc

N = 10000
E = 160000
D = 256
L = 5
G = 64
T = 128

RB = 256
NPAD = 10240
NTEC = 16
EB = 128
NB = 80
CH = 8
NCHUNK = NB // CH
EPAD = NTEC * NB * EB
RPT = NPAD // NTEC

@functools.cache
def _get_sc_aggregate():
    mesh = plsc.VectorSubcoreMesh(core_axis_name="c", subcore_axis_name="s")

    @functools.partial(
        pl.kernel,
        mesh=mesh,
        out_type=jax.ShapeDtypeStruct((2, NPAD, 128), jnp.float32),
        scratch_types=[
            pltpu.VMEM((EB,), jnp.int32),
            pltpu.VMEM((EB,), jnp.int32),
            pltpu.VMEM((EB,), jnp.int32),
            pltpu.VMEM((EB,), jnp.int32),
            pltpu.VMEM((EB, 128), jnp.float32),
            pltpu.VMEM((EB, 128), jnp.float32),
            pltpu.VMEM_SHARED((NPAD, 128), jnp.float32),
            pltpu.SemaphoreType.DMA,
            pltpu.SemaphoreType.DMA,
        ],
    )
    def sc_aggregate(h_hbm, src_hbm, dst_hbm, out_hbm,
                     sb0, sb1, db0, db1, bufa, bufb, acc, sga, sgb):
        core = lax.axis_index("c")
        s = lax.axis_index("s")
        base = s * RPT
        pltpu.sync_copy(h_hbm.at[core].at[pl.ds(base, RPT)],
                        acc.at[pl.ds(base, RPT)])
        plsc.subcore_barrier()

        def pair(p, carry):
            b = 2 * p
            pltpu.sync_copy(src_hbm.at[s].at[b], sb0)
            pltpu.sync_copy(src_hbm.at[s].at[b + 1], sb1)
            pltpu.sync_copy(dst_hbm.at[s].at[b], db0)
            pltpu.sync_copy(dst_hbm.at[s].at[b + 1], db1)
            g0 = pltpu.async_copy(h_hbm.at[core].at[sb0], bufa, sga)
            g1 = pltpu.async_copy(h_hbm.at[core].at[sb1], bufb, sgb)
            g0.wait()
            pltpu.sync_copy(bufa, acc.at[db0], add=True)
            g1.wait()
            pltpu.sync_copy(bufb, acc.at[db1], add=True)
            return carry

        lax.fori_loop(0, NB // 2, pair, 0)

        plsc.subcore_barrier()
        pltpu.sync_copy(acc.at[pl.ds(base, RPT)],
                        out_hbm.at[core].at[pl.ds(base, RPT)])

    return sc_aggregate


def _mlp_body(eps_ref, h_ref, hp_ref, w1_ref, b1_ref, w2_ref, b2_ref,
              out_ref, *, last):
    h = jnp.concatenate([h_ref[0], h_ref[1]], axis=-1)
    hp = jnp.concatenate([hp_ref[0], hp_ref[1]], axis=-1)
    z = hp + eps_ref[0, 0] * h
    z = jnp.maximum(
        jnp.dot(z, w1_ref[...], preferred_element_type=jnp.float32)
        + b1_ref[...], 0.0)
    hn = jnp.dot(z, w2_ref[...], preferred_element_type=jnp.float32) \
        + b2_ref[...]
    if not last:
        hn = jnp.maximum(hn, 0.0)
    out_ref[0] = hn[:, :128]
    out_ref[1] = hn[:, 128:]


def _mlp_call(eps_l, h, hp, w1, b1, w2, b2, last):
    return pl.pallas_call(
        functools.partial(_mlp_body, last=last),
        grid=(NPAD // RB,),
        in_specs=[
            pl.BlockSpec(memory_space=pltpu.SMEM),
            pl.BlockSpec((2, RB, 128), lambda i: (0, i, 0)),
            pl.BlockSpec((2, RB, 128), lambda i: (0, i, 0)),
            pl.BlockSpec((D, 2 * D), lambda i: (0, 0)),
            pl.BlockSpec((1, 2 * D), lambda i: (0, 0)),
            pl.BlockSpec((2 * D, D), lambda i: (0, 0)),
            pl.BlockSpec((1, D), lambda i: (0, 0)),
        ],
        out_specs=pl.BlockSpec((2, RB, 128), lambda i: (0, i, 0)),
        out_shape=jax.ShapeDtypeStruct((2, NPAD, 128), jnp.float32),
    )(eps_l, h, hp, w1, b1, w2, b2)


def _final_body(h_ref, b_ref, wn_ref, bn_ref, wp1_ref, bp1_ref,
                wp2_ref, bp2_ref, out_ref, seg, cnt):
    i = pl.program_id(0)

    @pl.when(i == 0)
    def _init():
        seg[...] = jnp.zeros_like(seg)
        cnt[...] = jnp.zeros_like(cnt)

    h = jnp.concatenate([h_ref[0], h_ref[1]], axis=-1)
    hn = jnp.maximum(
        jnp.dot(h, wn_ref[...], preferred_element_type=jnp.float32)
        + bn_ref[...], 0.0)
    bb = b_ref[0, 0, :].reshape(RB, 1)
    hn = jnp.where(bb < G, hn, 0.0)
    onehot = (bb == lax.broadcasted_iota(jnp.int32, (RB, G), 1)
              ).astype(jnp.float32)
    seg[...] += lax.dot_general(onehot, hn, (((0,), (0,)), ((), ())),
                                preferred_element_type=jnp.float32)
    cnt[...] += lax.dot_general(onehot, jnp.ones((RB, 1), jnp.float32),
                                (((0,), (0,)), ((), ())),
                                preferred_element_type=jnp.float32)

    @pl.when(i == NPAD // RB - 1)
    def _tail():
        g = seg[...] / jnp.maximum(cnt[...], 1.0)
        g = jnp.maximum(
            jnp.dot(g, wp1_ref[...], preferred_element_type=jnp.float32)
            + bp1_ref[...], 0.0)
        out_ref[...] = jnp.dot(g, wp2_ref[...],
                               preferred_element_type=jnp.float32) \
            + bp2_ref[...]


def _final_call(h, batch3d, wn, bn, wp1, bp1, wp2, bp2):
    return pl.pallas_call(
        _final_body,
        grid=(NPAD // RB,),
        in_specs=[
            pl.BlockSpec((2, RB, 128), lambda i: (0, i, 0)),
            pl.BlockSpec((1, 1, RB), lambda i: (i, 0, 0)),
            pl.BlockSpec((D, D), lambda i: (0, 0)),
            pl.BlockSpec((1, D), lambda i: (0, 0)),
            pl.BlockSpec((D, D), lambda i: (0, 0)),
            pl.BlockSpec((1, D), lambda i: (0, 0)),
            pl.BlockSpec((D, T), lambda i: (0, 0)),
            pl.BlockSpec((1, T), lambda i: (0, 0)),
        ],
        out_specs=pl.BlockSpec((G, T), lambda i: (0, 0)),
        out_shape=jax.ShapeDtypeStruct((G, T), jnp.float32),
        scratch_shapes=[
            pltpu.VMEM((G, D), jnp.float32),
            pltpu.VMEM((G, 1), jnp.float32),
        ],
    )(h, batch3d, wn, bn, wp1, bp1, wp2, bp2)


def kernel(x, edge_index, batch, W1, b1, W2, b2, eps, Wn, bn,
           Wp1, bp1, Wp2, bp2):
    src = edge_index[0]
    dst = edge_index[1]
    src_p = jnp.concatenate(
        [src, jnp.zeros((EPAD - E,), jnp.int32)]).reshape(NTEC, NB, EB)
    dst_p = jnp.concatenate(
        [dst, jnp.full((EPAD - E,), N, jnp.int32)]).reshape(NTEC, NB, EB)
    xp = jnp.pad(x, ((0, NPAD - N), (0, 0)))
    h = jnp.stack([xp[:, :128], xp[:, 128:]])
    batch3d = jnp.pad(batch, (0, NPAD - N), constant_values=G).reshape(
        NPAD // RB, 1, RB)

    sc_aggregate = _get_sc_aggregate()
    for l in range(L):
        hp = sc_aggregate(h, src_p, dst_p)
        h = _mlp_call(eps[l].reshape(1, 1), h, hp,
                      W1[l], b1[l].reshape(1, 2 * D),
                      W2[l], b2[l].reshape(1, D), last=(l == L - 1))

    return _final_call(h, batch3d, Wn, bn.reshape(1, D),
                       Wp1, bp1.reshape(1, D), Wp2, bp2.reshape(1, T))


# speedup vs baseline: 1.3306x; 1.3306x over previous
import functools

import jax
import jax.numpy as jnp
from jax import lax
from jax.experimental import pallas as pl
from jax.experimental.pallas import tpu as pltpu
from jax.experimental.pallas import tpu_sc as plsc

N = 10000
E = 160000
D = 256
L = 5
G = 64
T = 128

RB = 256
NPAD = 10240
NTEC = 16
EB = 128
NB = 79
EPAD = NTEC * NB * EB
RPT = NPAD // NTEC

@functools.cache
def _get_sc_aggregate():
    mesh = plsc.VectorSubcoreMesh(core_axis_name="c", subcore_axis_name="s")

    @functools.partial(
        pl.kernel,
        mesh=mesh,
        out_type=jax.ShapeDtypeStruct((2, NPAD, 128), jnp.float32),
        scratch_types=[
            pltpu.VMEM((EB,), jnp.int32),
            pltpu.VMEM((EB,), jnp.int32),
            pltpu.VMEM((EB, 128), jnp.float32),
            pltpu.VMEM_SHARED((NPAD, 128), jnp.float32),
            pltpu.SemaphoreType.DMA,
        ],
    )
    def sc_aggregate(h_hbm, src_hbm, dst_hbm, out_hbm,
                     sbuf, dbuf, bufa, acc, sga):
        core = lax.axis_index("c")
        s = lax.axis_index("s")
        base = s * RPT
        pltpu.sync_copy(h_hbm.at[core].at[pl.ds(base, RPT)],
                        acc.at[pl.ds(base, RPT)])
        plsc.subcore_barrier()

        def body(b, carry):
            pltpu.sync_copy(src_hbm.at[s].at[b], sbuf)
            pltpu.sync_copy(dst_hbm.at[s].at[b], dbuf)
            pltpu.async_copy(h_hbm.at[core].at[sbuf], bufa, sga).wait()
            pltpu.sync_copy(bufa, acc.at[dbuf], add=True)
            return carry

        lax.fori_loop(0, NB, body, 0)

        plsc.subcore_barrier()
        pltpu.sync_copy(acc.at[pl.ds(base, RPT)],
                        out_hbm.at[core].at[pl.ds(base, RPT)])

    return sc_aggregate


def _mlp_body(eps_ref, h_ref, hp_ref, w1_ref, b1_ref, w2_ref, b2_ref,
              out_ref, *, last):
    h = jnp.concatenate([h_ref[0], h_ref[1]], axis=-1)
    hp = jnp.concatenate([hp_ref[0], hp_ref[1]], axis=-1)
    z = hp + eps_ref[0, 0] * h
    z = jnp.maximum(
        jnp.dot(z, w1_ref[...], preferred_element_type=jnp.float32)
        + b1_ref[...], 0.0)
    hn = jnp.dot(z, w2_ref[...], preferred_element_type=jnp.float32) \
        + b2_ref[...]
    if not last:
        hn = jnp.maximum(hn, 0.0)
    out_ref[0] = hn[:, :128]
    out_ref[1] = hn[:, 128:]


def _mlp_call(eps_l, h, hp, w1, b1, w2, b2, last):
    return pl.pallas_call(
        functools.partial(_mlp_body, last=last),
        grid=(NPAD // RB,),
        in_specs=[
            pl.BlockSpec(memory_space=pltpu.SMEM),
            pl.BlockSpec((2, RB, 128), lambda i: (0, i, 0)),
            pl.BlockSpec((2, RB, 128), lambda i: (0, i, 0)),
            pl.BlockSpec((D, 2 * D), lambda i: (0, 0)),
            pl.BlockSpec((1, 2 * D), lambda i: (0, 0)),
            pl.BlockSpec((2 * D, D), lambda i: (0, 0)),
            pl.BlockSpec((1, D), lambda i: (0, 0)),
        ],
        out_specs=pl.BlockSpec((2, RB, 128), lambda i: (0, i, 0)),
        out_shape=jax.ShapeDtypeStruct((2, NPAD, 128), jnp.float32),
    )(eps_l, h, hp, w1, b1, w2, b2)


def _final_body(h_ref, b_ref, wn_ref, bn_ref, wp1_ref, bp1_ref,
                wp2_ref, bp2_ref, out_ref, seg, cnt):
    i = pl.program_id(0)

    @pl.when(i == 0)
    def _init():
        seg[...] = jnp.zeros_like(seg)
        cnt[...] = jnp.zeros_like(cnt)

    h = jnp.concatenate([h_ref[0], h_ref[1]], axis=-1)
    hn = jnp.maximum(
        jnp.dot(h, wn_ref[...], preferred_element_type=jnp.float32)
        + bn_ref[...], 0.0)
    bb = b_ref[0, 0, :].reshape(RB, 1)
    hn = jnp.where(bb < G, hn, 0.0)
    onehot = (bb == lax.broadcasted_iota(jnp.int32, (RB, G), 1)
              ).astype(jnp.float32)
    seg[...] += lax.dot_general(onehot, hn, (((0,), (0,)), ((), ())),
                                preferred_element_type=jnp.float32)
    cnt[...] += lax.dot_general(onehot, jnp.ones((RB, 1), jnp.float32),
                                (((0,), (0,)), ((), ())),
                                preferred_element_type=jnp.float32)

    @pl.when(i == NPAD // RB - 1)
    def _tail():
        g = seg[...] / jnp.maximum(cnt[...], 1.0)
        g = jnp.maximum(
            jnp.dot(g, wp1_ref[...], preferred_element_type=jnp.float32)
            + bp1_ref[...], 0.0)
        out_ref[...] = jnp.dot(g, wp2_ref[...],
                               preferred_element_type=jnp.float32) \
            + bp2_ref[...]


def _final_call(h, batch3d, wn, bn, wp1, bp1, wp2, bp2):
    return pl.pallas_call(
        _final_body,
        grid=(NPAD // RB,),
        in_specs=[
            pl.BlockSpec((2, RB, 128), lambda i: (0, i, 0)),
            pl.BlockSpec((1, 1, RB), lambda i: (i, 0, 0)),
            pl.BlockSpec((D, D), lambda i: (0, 0)),
            pl.BlockSpec((1, D), lambda i: (0, 0)),
            pl.BlockSpec((D, D), lambda i: (0, 0)),
            pl.BlockSpec((1, D), lambda i: (0, 0)),
            pl.BlockSpec((D, T), lambda i: (0, 0)),
            pl.BlockSpec((1, T), lambda i: (0, 0)),
        ],
        out_specs=pl.BlockSpec((G, T), lambda i: (0, 0)),
        out_shape=jax.ShapeDtypeStruct((G, T), jnp.float32),
        scratch_shapes=[
            pltpu.VMEM((G, D), jnp.float32),
            pltpu.VMEM((G, 1), jnp.float32),
        ],
    )(h, batch3d, wn, bn, wp1, bp1, wp2, bp2)


def kernel(x, edge_index, batch, W1, b1, W2, b2, eps, Wn, bn,
           Wp1, bp1, Wp2, bp2):
    src = edge_index[0]
    dst = edge_index[1]
    src_p = jnp.concatenate(
        [src, jnp.zeros((EPAD - E,), jnp.int32)]).reshape(NTEC, NB, EB)
    dst_p = jnp.concatenate(
        [dst, jnp.full((EPAD - E,), N, jnp.int32)]).reshape(NTEC, NB, EB)
    xp = jnp.pad(x, ((0, NPAD - N), (0, 0)))
    h = jnp.stack([xp[:, :128], xp[:, 128:]])
    batch3d = jnp.pad(batch, (0, NPAD - N), constant_values=G).reshape(
        NPAD // RB, 1, RB)

    sc_aggregate = _get_sc_aggregate()
    for l in range(L):
        hp = sc_aggregate(h, src_p, dst_p)
        h = _mlp_call(eps[l].reshape(1, 1), h, hp,
                      W1[l], b1[l].reshape(1, 2 * D),
                      W2[l], b2[l].reshape(1, D), last=(l == L - 1))

    return _final_call(h, batch3d, Wn, bn.reshape(1, D),
                       Wp1, bp1.reshape(1, D), Wp2, bp2.reshape(1, T))
